# Initial kernel scaffold; baseline (speedup 1.0000x reference)
import jax
import jax.numpy as jnp
from jax.experimental import pallas as pl


def kernel(x, weights, top_weights, top_experts, W1, V1, W2):
    raise NotImplementedError("write your pallas kernel here")


# speedup vs baseline: 1.2508x; 1.2508x over previous
import functools

import jax
import jax.numpy as jnp
from jax.experimental import pallas as pl
from jax.experimental.pallas import tpu as pltpu

E = 8
TOPK = 2
D = 1024
FFN = 4096
T = 2048
P = T * TOPK
TM = 256
NT = 24
NPAD = NT * TM
BF = 1024
J = FFN // BF


def _route_host(top_experts, top_weights):
    ef = top_experts.reshape(P).astype(jnp.int32)
    counts = jnp.bincount(ef, length=E)
    cstart = jnp.concatenate([jnp.zeros(1, jnp.int32),
                              jnp.cumsum(counts)[:-1].astype(jnp.int32)])
    aligned = ((counts + TM - 1) // TM) * TM
    astart = jnp.concatenate([jnp.zeros(1, jnp.int32),
                              jnp.cumsum(aligned)[:-1].astype(jnp.int32)])
    order = jnp.argsort(ef, stable=True)
    e_of_c = ef[order]
    apos = astart[e_of_c] + (jnp.arange(P, dtype=jnp.int32) - cstart[e_of_c])
    perm = jnp.zeros(NPAD, jnp.int32).at[apos].set((order // TOPK).astype(jnp.int32))
    wsort = jnp.zeros(NPAD, jnp.float32).at[apos].set(top_weights.reshape(P)[order])
    inv = jnp.zeros(P, jnp.int32).at[order].set(apos)
    ends = ((astart + aligned) // TM).astype(jnp.int32)
    eid = jnp.searchsorted(ends, jnp.arange(NT, dtype=jnp.int32), side="right")
    eid = jnp.minimum(eid, E - 1).astype(jnp.int32)
    nact = ends[-1]
    meta = jnp.concatenate([eid, nact[None].astype(jnp.int32)])
    return perm, wsort, inv, meta


def _glu_body(meta_ref, x_ref, w1_ref, v1_ref, w2_ref, ws_ref, out_ref):
    t = pl.program_id(0)
    j = pl.program_id(1)
    nact = meta_ref[NT]

    @pl.when(j == 0)
    def _():
        out_ref[...] = jnp.zeros_like(out_ref)

    @pl.when(t < nact)
    def _():
        x = x_ref[...]
        w1 = w1_ref[0]
        v1 = v1_ref[0]
        w2 = w2_ref[0]
        gate = jax.lax.dot_general(x, w1, (((1,), (1,)), ((), ())),
                                   preferred_element_type=jnp.float32)
        up = jax.lax.dot_general(x, v1, (((1,), (1,)), ((), ())),
                                 preferred_element_type=jnp.float32)
        inter = (gate * jax.lax.logistic(gate)) * up
        part = jax.lax.dot_general(inter, w2, (((1,), (0,)), ((), ())),
                                   preferred_element_type=jnp.float32)
        out_ref[...] += part * ws_ref[...]


def _glu_grouped(meta, xs, W1, V1, W2, wsort):
    grid_spec = pltpu.PrefetchScalarGridSpec(
        num_scalar_prefetch=1,
        grid=(NT, J),
        in_specs=[
            pl.BlockSpec((TM, D), lambda t, j, m: (t, 0)),
            pl.BlockSpec((1, BF, D), lambda t, j, m: (m[t], j, 0)),
            pl.BlockSpec((1, BF, D), lambda t, j, m: (m[t], j, 0)),
            pl.BlockSpec((1, BF, D), lambda t, j, m: (m[t], j, 0)),
            pl.BlockSpec((TM, 1), lambda t, j, m: (t, 0)),
        ],
        out_specs=pl.BlockSpec((TM, D), lambda t, j, m: (t, 0)),
    )
    return pl.pallas_call(
        _glu_body,
        grid_spec=grid_spec,
        out_shape=jax.ShapeDtypeStruct((NPAD, D), jnp.float32),
        compiler_params=pltpu.CompilerParams(
            dimension_semantics=("arbitrary", "arbitrary")),
    )(meta, xs, W1, V1, W2, wsort.reshape(NPAD, 1))


def kernel(x, weights, top_weights, top_experts, W1, V1, W2):
    xf = x.reshape(T, D)
    top_experts = top_experts.astype(jnp.int32)
    perm, wsort, inv, meta = _route_host(top_experts, top_weights)
    xs = xf[perm]
    ys = _glu_grouped(meta, xs, W1, V1, W2, wsort)
    inv2 = inv.reshape(T, TOPK)
    out = ys[inv2[:, 0]] + ys[inv2[:, 1]]
    return out.reshape(x.shape)


# speedup vs baseline: 1.4165x; 1.1325x over previous
import functools

import jax
import jax.numpy as jnp
from jax import lax
from jax.experimental import pallas as pl
from jax.experimental.pallas import tpu as pltpu
from jax.experimental.pallas import tpu_sc as plsc

E = 8
TOPK = 2
D = 1024
FFN = 4096
T = 2048
P = T * TOPK
TM = 256
NT = 24
NPAD = NT * TM
BF = 1024
J = FFN // BF


_LANES = 16
_TMSHIFT = 8


def _route_body(te_hbm, tw_hbm, perm_hbm, wsort_hbm, inv_hbm, meta_hbm,
                te_v, tw_v, perm_v, wsort_v, inv_v, cur_v, endt_v, meta_v):
    cid = lax.axis_index("c")
    sid = lax.axis_index("s")
    wid = sid * 2 + cid

    @pl.when(wid == 0)
    def _():
        pltpu.sync_copy(te_hbm, te_v)
        pltpu.sync_copy(tw_hbm, tw_v)
        lanes = lax.iota(jnp.int32, _LANES)
        zi = jnp.zeros((_LANES,), jnp.int32)
        zf = jnp.zeros((_LANES,), jnp.float32)

        def zbody(i, carry):
            perm_v[pl.ds(i * _LANES, _LANES)] = zi
            wsort_v[pl.ds(i * _LANES, _LANES)] = zf
            return carry

        lax.fori_loop(0, NPAD // _LANES, zbody, 0)

        def hbody(c, cnt):
            ev = te_v[pl.ds(c * _LANES, _LANES)]
            for b in range(E):
                cs = plsc.cumsum(jnp.where(ev == b, 1, 0))
                cnt = cnt + jnp.where(lanes == b, jnp.max(cs), 0)
            return cnt

        cnt = lax.fori_loop(0, P // _LANES, hbody, zi)

        aligned = ((cnt + (TM - 1)) >> _TMSHIFT) << _TMSHIFT
        incl = plsc.cumsum(aligned)
        cur_v[...] = incl - aligned
        endt_v[...] = incl >> _TMSHIFT
        endt = endt_v[...]
        acc0 = zi
        acc1 = zi
        tv1 = lanes + _LANES
        for e in range(E):
            et = endt[e]
            acc0 = acc0 + jnp.where(lanes >= et, 1, 0)
            acc1 = acc1 + jnp.where(tv1 >= et, 1, 0)
        nact = endt[E - 1]
        meta_v[pl.ds(0, _LANES)] = jnp.minimum(acc0, E - 1)
        meta_v[pl.ds(_LANES, _LANES)] = jnp.where(tv1 == NT, nact,
                                                  jnp.minimum(acc1, E - 1))
        pltpu.sync_copy(meta_v, meta_hbm)

        ones = jnp.ones((_LANES,), jnp.int32)

        def sbody(c, carry):
            ev = te_v[pl.ds(c * _LANES, _LANES)]
            twv = tw_v[pl.ds(c * _LANES, _LANES)]
            base = plsc.load_gather(cur_v, [ev])
            rank = zi
            add = zi
            for b in range(E):
                m = ev == b
                cs = plsc.cumsum(jnp.where(m, 1, 0))
                rank = rank + jnp.where(m, cs - 1, 0)
                add = add + jnp.where(lanes == b, jnp.max(cs), 0)
            pos = base + rank
            tok = (lanes + c * _LANES) >> 1
            plsc.store_scatter(perm_v, [pos], tok)
            plsc.store_scatter(wsort_v, [pos], twv)
            inv_v[pl.ds(c * _LANES, _LANES)] = pos
            cur_v[...] = cur_v[...] + add
            return carry

        lax.fori_loop(0, P // _LANES, sbody, 0)
        pltpu.sync_copy(perm_v, perm_hbm)
        pltpu.sync_copy(wsort_v, wsort_hbm)
        pltpu.sync_copy(inv_v, inv_hbm)


def _route_sc(top_experts, top_weights):
    te = top_experts.reshape(P).astype(jnp.int32)
    tw = top_weights.reshape(P).astype(jnp.float32)
    mesh = plsc.VectorSubcoreMesh(core_axis_name="c", subcore_axis_name="s")
    perm, wsort, inv, meta = pl.kernel(
        _route_body,
        out_type=(
            jax.ShapeDtypeStruct((NPAD,), jnp.int32),
            jax.ShapeDtypeStruct((NPAD,), jnp.float32),
            jax.ShapeDtypeStruct((P,), jnp.int32),
            jax.ShapeDtypeStruct((2 * _LANES,), jnp.int32),
        ),
        mesh=mesh,
        scratch_types=[
            pltpu.VMEM((P,), jnp.int32),
            pltpu.VMEM((P,), jnp.float32),
            pltpu.VMEM((NPAD,), jnp.int32),
            pltpu.VMEM((NPAD,), jnp.float32),
            pltpu.VMEM((P,), jnp.int32),
            pltpu.VMEM((_LANES,), jnp.int32),
            pltpu.VMEM((_LANES,), jnp.int32),
            pltpu.VMEM((2 * _LANES,), jnp.int32),
        ],
        compiler_params=pltpu.CompilerParams(needs_layout_passes=False),
    )(te, tw)
    return perm, wsort, inv, meta[: NT + 1]


def _route_host(top_experts, top_weights):
    ef = top_experts.reshape(P).astype(jnp.int32)
    counts = jnp.bincount(ef, length=E)
    cstart = jnp.concatenate([jnp.zeros(1, jnp.int32),
                              jnp.cumsum(counts)[:-1].astype(jnp.int32)])
    aligned = ((counts + TM - 1) // TM) * TM
    astart = jnp.concatenate([jnp.zeros(1, jnp.int32),
                              jnp.cumsum(aligned)[:-1].astype(jnp.int32)])
    order = jnp.argsort(ef, stable=True)
    e_of_c = ef[order]
    apos = astart[e_of_c] + (jnp.arange(P, dtype=jnp.int32) - cstart[e_of_c])
    perm = jnp.zeros(NPAD, jnp.int32).at[apos].set((order // TOPK).astype(jnp.int32))
    wsort = jnp.zeros(NPAD, jnp.float32).at[apos].set(top_weights.reshape(P)[order])
    inv = jnp.zeros(P, jnp.int32).at[order].set(apos)
    ends = ((astart + aligned) // TM).astype(jnp.int32)
    eid = jnp.searchsorted(ends, jnp.arange(NT, dtype=jnp.int32), side="right")
    eid = jnp.minimum(eid, E - 1).astype(jnp.int32)
    nact = ends[-1]
    meta = jnp.concatenate([eid, nact[None].astype(jnp.int32)])
    return perm, wsort, inv, meta


def _glu_body(meta_ref, x_ref, w1_ref, v1_ref, w2_ref, ws_ref, out_ref):
    t = pl.program_id(0)
    j = pl.program_id(1)
    nact = meta_ref[NT]

    @pl.when(j == 0)
    def _():
        out_ref[...] = jnp.zeros_like(out_ref)

    @pl.when(t < nact)
    def _():
        x = x_ref[...]
        w1 = w1_ref[0]
        v1 = v1_ref[0]
        w2 = w2_ref[0]
        gate = jax.lax.dot_general(x, w1, (((1,), (1,)), ((), ())),
                                   preferred_element_type=jnp.float32)
        up = jax.lax.dot_general(x, v1, (((1,), (1,)), ((), ())),
                                 preferred_element_type=jnp.float32)
        inter = (gate * jax.lax.logistic(gate)) * up
        part = jax.lax.dot_general(inter, w2, (((1,), (0,)), ((), ())),
                                   preferred_element_type=jnp.float32)
        out_ref[...] += part * ws_ref[...]


def _glu_grouped(meta, xs, W1, V1, W2, wsort):
    grid_spec = pltpu.PrefetchScalarGridSpec(
        num_scalar_prefetch=1,
        grid=(NT, J),
        in_specs=[
            pl.BlockSpec((TM, D), lambda t, j, m: (t, 0)),
            pl.BlockSpec((1, BF, D), lambda t, j, m: (m[t], j, 0)),
            pl.BlockSpec((1, BF, D), lambda t, j, m: (m[t], j, 0)),
            pl.BlockSpec((1, BF, D), lambda t, j, m: (m[t], j, 0)),
            pl.BlockSpec((TM, 1), lambda t, j, m: (t, 0)),
        ],
        out_specs=pl.BlockSpec((TM, D), lambda t, j, m: (t, 0)),
    )
    return pl.pallas_call(
        _glu_body,
        grid_spec=grid_spec,
        out_shape=jax.ShapeDtypeStruct((NPAD, D), jnp.float32),
        compiler_params=pltpu.CompilerParams(
            dimension_semantics=("arbitrary", "arbitrary")),
    )(meta, xs, W1, V1, W2, wsort.reshape(NPAD, 1))


def kernel(x, weights, top_weights, top_experts, W1, V1, W2):
    xf = x.reshape(T, D)
    top_experts = top_experts.astype(jnp.int32)
    perm, wsort, inv, meta = _route_sc(top_experts, top_weights)
    xs = xf[perm]
    ys = _glu_grouped(meta, xs, W1, V1, W2, wsort)
    inv2 = inv.reshape(T, TOPK)
    out = ys[inv2[:, 0]] + ys[inv2[:, 1]]
    return out.reshape(x.shape)
